# baseline (speedup 1.0000x reference)
import functools

import jax
import jax.numpy as jnp
from jax import lax
from jax.experimental import pallas as pl
from jax.experimental.pallas import tpu as pltpu
from jax.experimental.pallas import tpu_sc as plsc

NUM_EMB = 8192
DIM = 64
N_TOK = 9216
COMMIT = 0.25

TB = 512
NB = N_TOK // TB

NW = 32
B_PER_W = N_TOK // NW
IDX_CHUNK = 96
N_CHUNK = B_PER_W // IDX_CHUNK


def _vq_block(x_ref, embT_ref, esq_ref, xsq_ref,
              enc_ref, idx_ref, loss_ref, perp_ref,
              acc_ref, cnt_ref):
    i = pl.program_id(0)

    @pl.when(i == 0)
    def _init():
        cnt_ref[...] = jnp.zeros_like(cnt_ref)
        acc_ref[0] = 0.0

    xb = x_ref[...].astype(jnp.bfloat16)
    dot = lax.dot_general(xb, embT_ref[...],
                          (((1,), (0,)), ((), ())),
                          preferred_element_type=jnp.float32)
    d = (xsq_ref[...] + esq_ref[...]) - dot
    dmin = jnp.min(d, axis=1, keepdims=True)
    iota = lax.broadcasted_iota(jnp.int32, (TB, NUM_EMB), 1)
    idxv = jnp.min(jnp.where(d == dmin, iota, NUM_EMB),
                   axis=1, keepdims=True)
    m = iota == idxv
    enc_ref[...] = m.astype(jnp.float32)
    idx_ref[...] = idxv
    ones_row = jnp.ones((8, TB), jnp.bfloat16)
    csum = lax.dot_general(ones_row, m.astype(jnp.bfloat16),
                           (((1,), (0,)), ((), ())),
                           preferred_element_type=jnp.float32)
    cnt_ref[...] += csum[:1]
    acc_ref[0] += jnp.sum(dmin)

    @pl.when(i == NB - 1)
    def _fin():
        loss_ref[...] = (COMMIT * (acc_ref[0] / (N_TOK * DIM))).reshape(1, 1)
        p = cnt_ref[...] * (1.0 / N_TOK)
        perp_ref[...] = jnp.exp(-jnp.sum(p * jnp.log(p + 1e-10))).reshape(1, 1)


def _tc_pass(flat, embT_bf16, esq_row, xsq_col):
    return pl.pallas_call(
        _vq_block,
        grid=(NB,),
        in_specs=[
            pl.BlockSpec((TB, DIM), lambda i: (i, 0)),
            pl.BlockSpec((DIM, NUM_EMB), lambda i: (0, 0)),
            pl.BlockSpec((1, NUM_EMB), lambda i: (0, 0)),
            pl.BlockSpec((TB, 1), lambda i: (i, 0)),
        ],
        out_specs=[
            pl.BlockSpec((TB, NUM_EMB), lambda i: (i, 0)),
            pl.BlockSpec((TB, 1), lambda i: (i, 0)),
            pl.BlockSpec((1, 1), lambda i: (0, 0)),
            pl.BlockSpec((1, 1), lambda i: (0, 0)),
        ],
        out_shape=[
            jax.ShapeDtypeStruct((N_TOK, NUM_EMB), jnp.float32),
            jax.ShapeDtypeStruct((N_TOK, 1), jnp.int32),
            jax.ShapeDtypeStruct((1, 1), jnp.float32),
            jax.ShapeDtypeStruct((1, 1), jnp.float32),
        ],
        scratch_shapes=[
            pltpu.SMEM((1,), jnp.float32),
            pltpu.VMEM((1, NUM_EMB), jnp.float32),
        ],
        compiler_params=pltpu.CompilerParams(
            vmem_limit_bytes=100 * 1024 * 1024),
    )(flat, embT_bf16, esq_row, xsq_col)


PAD_D = 128


def _sc_gather(table_pad, idx_flat):
    mesh = plsc.VectorSubcoreMesh(core_axis_name="c", subcore_axis_name="s")

    @functools.partial(
        pl.kernel,
        mesh=mesh,
        out_type=jax.ShapeDtypeStruct((N_TOK, PAD_D), jnp.float32),
        scratch_types=[
            pltpu.VMEM((B_PER_W,), jnp.int32),
            pltpu.VMEM((B_PER_W, PAD_D), jnp.float32),
            pltpu.SemaphoreType.DMA,
        ],
    )
    def k(table_hbm, idx_hbm, out_hbm, idx_v, rows_v, sem):
        wid = lax.axis_index("s") * 2 + lax.axis_index("c")
        pltpu.sync_copy(idx_hbm.at[pl.ds(wid * B_PER_W, B_PER_W)], idx_v)
        for j in range(N_CHUNK):
            pltpu.async_copy(table_hbm.at[idx_v.at[pl.ds(j * IDX_CHUNK, IDX_CHUNK)]],
                             rows_v.at[pl.ds(j * IDX_CHUNK, IDX_CHUNK)],
                             sem).wait()
        pltpu.sync_copy(rows_v, out_hbm.at[pl.ds(wid * B_PER_W, B_PER_W)])

    return k(table_pad, idx_flat)


def kernel(inputs, embedding):
    flat = inputs.reshape(-1, DIM)
    xsq = jnp.sum(flat ** 2, axis=1, keepdims=True)
    esq = jnp.sum(embedding ** 2, axis=1)
    embT = (2.0 * embedding).T.astype(jnp.bfloat16)

    enc, idx, loss2, perp2 = _tc_pass(flat, embT, esq.reshape(1, NUM_EMB), xsq)

    table_pad = jnp.pad(embedding, ((0, 0), (0, PAD_D - DIM)))
    quant_pad = _sc_gather(table_pad, idx.reshape(N_TOK))
    quant = quant_pad[:, :DIM].reshape(inputs.shape)

    return loss2[0, 0], quant, perp2[0, 0], enc


# speedup vs baseline: 1.1359x; 1.1359x over previous
import functools

import jax
import jax.numpy as jnp
from jax import lax
from jax.experimental import pallas as pl
from jax.experimental.pallas import tpu as pltpu
from jax.experimental.pallas import tpu_sc as plsc

NUM_EMB = 8192
DIM = 64
N_TOK = 9216
COMMIT = 0.25

TB = 512
NB = N_TOK // TB

NW = 32
B_PER_W = N_TOK // NW
IDX_CHUNK = 96
N_CHUNK = B_PER_W // IDX_CHUNK


def _vq_block(x_ref, embT_ref, esq_ref, xsq_ref,
              enc_ref, idx_ref, loss_ref, perp_ref,
              acc_ref, cnt_ref):
    i = pl.program_id(0)

    @pl.when(i == 0)
    def _init():
        cnt_ref[...] = jnp.zeros_like(cnt_ref)
        acc_ref[0] = 0.0

    xb = x_ref[...].astype(jnp.bfloat16)
    dot = lax.dot_general(xb, embT_ref[...],
                          (((1,), (0,)), ((), ())),
                          preferred_element_type=jnp.float32)
    d = (xsq_ref[...] + esq_ref[...]) - 2.0 * dot
    dmin = jnp.min(d, axis=1, keepdims=True)
    iota = lax.broadcasted_iota(jnp.int32, (TB, NUM_EMB), 1)
    idxv = jnp.min(jnp.where(d == dmin, iota, NUM_EMB),
                   axis=1, keepdims=True)
    m = iota == idxv
    enc_ref[...] = m.astype(jnp.float32)
    idx_ref[...] = idxv
    ones_row = jnp.ones((8, TB), jnp.bfloat16)
    csum = lax.dot_general(ones_row, m.astype(jnp.bfloat16),
                           (((1,), (0,)), ((), ())),
                           preferred_element_type=jnp.float32)
    cnt_ref[...] += csum[:1]
    acc_ref[0] += jnp.sum(dmin)

    @pl.when(i == NB - 1)
    def _fin():
        loss_ref[...] = (COMMIT * (acc_ref[0] / (N_TOK * DIM))).reshape(1, 1)
        p = cnt_ref[...] * (1.0 / N_TOK)
        perp_ref[...] = jnp.exp(-jnp.sum(p * jnp.log(p + 1e-10))).reshape(1, 1)


def _tc_pass(flat, embT_bf16, esq_row, xsq_col):
    return pl.pallas_call(
        _vq_block,
        grid=(NB,),
        in_specs=[
            pl.BlockSpec((TB, DIM), lambda i: (i, 0)),
            pl.BlockSpec((DIM, NUM_EMB), lambda i: (0, 0)),
            pl.BlockSpec((1, NUM_EMB), lambda i: (0, 0)),
            pl.BlockSpec((TB, 1), lambda i: (i, 0)),
        ],
        out_specs=[
            pl.BlockSpec((TB, NUM_EMB), lambda i: (i, 0)),
            pl.BlockSpec((TB, 1), lambda i: (i, 0)),
            pl.BlockSpec((1, 1), lambda i: (0, 0)),
            pl.BlockSpec((1, 1), lambda i: (0, 0)),
        ],
        out_shape=[
            jax.ShapeDtypeStruct((N_TOK, NUM_EMB), jnp.float32),
            jax.ShapeDtypeStruct((N_TOK, 1), jnp.int32),
            jax.ShapeDtypeStruct((1, 1), jnp.float32),
            jax.ShapeDtypeStruct((1, 1), jnp.float32),
        ],
        scratch_shapes=[
            pltpu.SMEM((1,), jnp.float32),
            pltpu.VMEM((1, NUM_EMB), jnp.float32),
        ],
        compiler_params=pltpu.CompilerParams(
            vmem_limit_bytes=100 * 1024 * 1024),
    )(flat, embT_bf16, esq_row, xsq_col)


PAD_D = 128


def _sc_gather(table_pad, idx_flat):
    mesh = plsc.VectorSubcoreMesh(core_axis_name="c", subcore_axis_name="s")

    @functools.partial(
        pl.kernel,
        mesh=mesh,
        out_type=jax.ShapeDtypeStruct((N_TOK, PAD_D), jnp.float32),
        scratch_types=[
            pltpu.VMEM((B_PER_W,), jnp.int32),
            pltpu.VMEM((B_PER_W, PAD_D), jnp.float32),
            pltpu.SemaphoreType.DMA,
        ],
    )
    def k(table_hbm, idx_hbm, out_hbm, idx_v, rows_v, sem):
        wid = lax.axis_index("s") * 2 + lax.axis_index("c")
        pltpu.sync_copy(idx_hbm.at[pl.ds(wid * B_PER_W, B_PER_W)], idx_v)
        for j in range(N_CHUNK):
            pltpu.async_copy(table_hbm.at[idx_v.at[pl.ds(j * IDX_CHUNK, IDX_CHUNK)]],
                             rows_v.at[pl.ds(j * IDX_CHUNK, IDX_CHUNK)],
                             sem).wait()
        pltpu.sync_copy(rows_v, out_hbm.at[pl.ds(wid * B_PER_W, B_PER_W)])

    return k(table_pad, idx_flat)


def kernel(inputs, embedding):
    flat = inputs.reshape(-1, DIM)
    xsq = jnp.sum(flat ** 2, axis=1, keepdims=True)
    esq = jnp.sum(embedding ** 2, axis=1)
    embT = embedding.T.astype(jnp.bfloat16)

    enc, idx, loss2, perp2 = _tc_pass(flat, embT, esq.reshape(1, NUM_EMB), xsq)

    table_pad = jnp.pad(embedding, ((0, 0), (0, PAD_D - DIM)))
    quant_pad = _sc_gather(table_pad, idx.reshape(N_TOK))
    quant = quant_pad[:, :DIM].reshape(inputs.shape)

    return loss2[0, 0], quant, perp2[0, 0], enc
